# baseline (speedup 1.0000x reference)
import functools

import jax
import jax.numpy as jnp
from jax import lax
from jax.experimental import pallas as pl
from jax.experimental.pallas import tpu as pltpu
from jax.experimental.pallas import tpu_sc as plsc

NUM_STATES = 1000000
EMBEDDING_DIM = 64
BATCH = 16384

_info = plsc.get_sparse_core_info()
_NC, _NS, _L = _info.num_cores, _info.num_subcores, _info.num_lanes
_NW = _NC * _NS
_B_PER_W = BATCH // _NW
_C_ROWS = 32
_NCHUNK = _B_PER_W // _C_ROWS

_TW = 2048


def _transpose_block(eye_ref, in_ref, out_ref):
    y = lax.dot_general(
        in_ref[...].astype(jnp.bfloat16),
        eye_ref[...],
        dimension_numbers=(((0,), (0,)), ((), ())),
        preferred_element_type=jnp.float32,
    )
    bits = lax.bitcast_convert_type(y, jnp.int32)
    half = EMBEDDING_DIM // 2
    lo = bits[:, :half]
    hi = bits[:, half:]
    out_ref[...] = ((lo >> 16) & 0xFFFF) | (hi & jnp.int32(-65536))


_tc_transpose = pl.pallas_call(
    _transpose_block,
    grid=(pl.cdiv(NUM_STATES, _TW),),
    in_specs=[
        pl.BlockSpec((EMBEDDING_DIM, EMBEDDING_DIM), lambda i: (0, 0)),
        pl.BlockSpec((EMBEDDING_DIM, _TW), lambda i: (0, i)),
    ],
    out_specs=pl.BlockSpec((_TW, EMBEDDING_DIM // 2), lambda i: (i, 0)),
    out_shape=jax.ShapeDtypeStruct((NUM_STATES, EMBEDDING_DIM // 2), jnp.int32),
)


def _make_gather():
    mesh = plsc.VectorSubcoreMesh(core_axis_name="c", subcore_axis_name="s")

    @functools.partial(
        pl.kernel,
        mesh=mesh,
        out_type=jax.ShapeDtypeStruct((BATCH, EMBEDDING_DIM // 2), jnp.int32),
        scratch_types=[
            pltpu.VMEM((_B_PER_W,), jnp.int32),
            pltpu.VMEM((2, _C_ROWS, EMBEDDING_DIM // 2), jnp.int32),
            [pltpu.SemaphoreType.DMA] * 2,
        ],
    )
    def gather_kernel(table_hbm, idx_hbm, out_hbm, idx_v, rbuf, sems):
        wid = lax.axis_index("s") * _NC + lax.axis_index("c")
        base = wid * _B_PER_W
        pltpu.sync_copy(idx_hbm.at[pl.ds(base, _B_PER_W)], idx_v)

        def issue_chunk(j):
            p = j % 2
            for h in range(_C_ROWS // _L):
                v = idx_v[pl.ds(j * _C_ROWS + h * _L, _L)]
                for l in range(_L):
                    pltpu.async_copy(
                        table_hbm.at[v[l]],
                        rbuf.at[p, h * _L + l],
                        sems[p],
                    )

        def drain_and_writeback(j):
            p = j % 2
            dst = out_hbm.at[pl.ds(base + j * _C_ROWS, _C_ROWS)]
            pltpu.make_async_copy(dst, rbuf.at[p], sems[p]).wait()
            pltpu.sync_copy(rbuf.at[p], dst)

        issue_chunk(0)
        for j in range(1, _NCHUNK):
            issue_chunk(j)
            drain_and_writeback(j - 1)
        drain_and_writeback(_NCHUNK - 1)

    return gather_kernel


_gather = _make_gather()


def kernel(state_id, state_embedding):
    eye = jnp.eye(EMBEDDING_DIM, dtype=jnp.bfloat16)
    table_rm = _tc_transpose(eye, state_embedding.T)
    out_i32 = _gather(table_rm, state_id.astype(jnp.int32))
    f_lo = lax.bitcast_convert_type(out_i32 << 16, jnp.float32)
    f_hi = lax.bitcast_convert_type(out_i32 & jnp.int32(-65536), jnp.float32)
    return jnp.concatenate([f_lo, f_hi], axis=1)


# speedup vs baseline: 1.6997x; 1.6997x over previous
import functools

import jax
import jax.numpy as jnp
from jax import lax
from jax.experimental import pallas as pl
from jax.experimental.pallas import tpu as pltpu
from jax.experimental.pallas import tpu_sc as plsc

NUM_STATES = 1000000
EMBEDDING_DIM = 64
BATCH = 16384

_info = plsc.get_sparse_core_info()
_NC, _NS, _L = _info.num_cores, _info.num_subcores, _info.num_lanes
_NW = _NC * _NS
_B_PER_W = BATCH // _NW
_C_ROWS = 32
_NCHUNK = _B_PER_W // _C_ROWS

_TW = 8192


def _transpose_block(eye_ref, in_ref, out_ref):
    y = lax.dot_general(
        in_ref[...].astype(jnp.bfloat16),
        eye_ref[...],
        dimension_numbers=(((0,), (0,)), ((), ())),
        preferred_element_type=jnp.float32,
    )
    bits = lax.bitcast_convert_type(y, jnp.int32)
    half = EMBEDDING_DIM // 2
    lo = bits[:, :half]
    hi = bits[:, half:]
    out_ref[...] = ((lo >> 16) & 0xFFFF) | (hi & jnp.int32(-65536))


_tc_transpose = pl.pallas_call(
    _transpose_block,
    grid=(pl.cdiv(NUM_STATES, _TW),),
    in_specs=[
        pl.BlockSpec((EMBEDDING_DIM, EMBEDDING_DIM), lambda i: (0, 0)),
        pl.BlockSpec((EMBEDDING_DIM, _TW), lambda i: (0, i)),
    ],
    out_specs=pl.BlockSpec((_TW, EMBEDDING_DIM // 2), lambda i: (i, 0)),
    out_shape=jax.ShapeDtypeStruct((NUM_STATES, EMBEDDING_DIM // 2), jnp.int32),
)


def _make_gather():
    mesh = plsc.VectorSubcoreMesh(core_axis_name="c", subcore_axis_name="s")

    @functools.partial(
        pl.kernel,
        mesh=mesh,
        out_type=jax.ShapeDtypeStruct((BATCH, EMBEDDING_DIM // 2), jnp.int32),
        scratch_types=[
            pltpu.VMEM((_B_PER_W,), jnp.int32),
            pltpu.VMEM((2, _C_ROWS, EMBEDDING_DIM // 2), jnp.int32),
            [pltpu.SemaphoreType.DMA] * 2,
        ],
    )
    def gather_kernel(table_hbm, idx_hbm, out_hbm, idx_v, rbuf, sems):
        wid = lax.axis_index("s") * _NC + lax.axis_index("c")
        base = wid * _B_PER_W
        pltpu.sync_copy(idx_hbm.at[pl.ds(base, _B_PER_W)], idx_v)

        def issue_chunk(j):
            p = j % 2
            for h in range(_C_ROWS // _L):
                v = idx_v[pl.ds(j * _C_ROWS + h * _L, _L)]
                for l in range(_L):
                    pltpu.async_copy(
                        table_hbm.at[v[l]],
                        rbuf.at[p, h * _L + l],
                        sems[p],
                    )

        def drain_and_writeback(j):
            p = j % 2
            dst = out_hbm.at[pl.ds(base + j * _C_ROWS, _C_ROWS)]
            pltpu.make_async_copy(dst, rbuf.at[p], sems[p]).wait()
            pltpu.sync_copy(rbuf.at[p], dst)

        issue_chunk(0)
        for j in range(1, _NCHUNK):
            issue_chunk(j)
            drain_and_writeback(j - 1)
        drain_and_writeback(_NCHUNK - 1)

    return gather_kernel


_gather = _make_gather()


def kernel(state_id, state_embedding):
    eye = jnp.eye(EMBEDDING_DIM, dtype=jnp.bfloat16)
    table_rm = _tc_transpose(eye, state_embedding.T)
    out_i32 = _gather(table_rm, state_id.astype(jnp.int32))
    f_lo = lax.bitcast_convert_type(out_i32 << 16, jnp.float32)
    f_hi = lax.bitcast_convert_type(out_i32 & jnp.int32(-65536), jnp.float32)
    return jnp.concatenate([f_lo, f_hi], axis=1)


# speedup vs baseline: 2.4083x; 1.4169x over previous
import functools

import jax
import jax.numpy as jnp
from jax import lax
from jax.experimental import pallas as pl
from jax.experimental.pallas import tpu as pltpu
from jax.experimental.pallas import tpu_sc as plsc

NUM_STATES = 1000000
EMBEDDING_DIM = 64
BATCH = 16384

_info = plsc.get_sparse_core_info()
_NC, _NS, _L = _info.num_cores, _info.num_subcores, _info.num_lanes
_NW = _NC * _NS
_B_PER_W = BATCH // _NW
_C_ROWS = 32
_NCHUNK = _B_PER_W // _C_ROWS

_H = 499968
_TW = 16128
_NB = _H // _TW
_PK_ROWS = _H + EMBEDDING_DIM


def _transpose_block(eye_ref, in_a_ref, in_b_ref, out_ref):
    def tr(x):
        return lax.dot_general(
            x.astype(jnp.bfloat16),
            eye_ref[...],
            dimension_numbers=(((0,), (0,)), ((), ())),
            preferred_element_type=jnp.float32,
        )

    out_ref[...] = jnp.concatenate([tr(in_a_ref[...]), tr(in_b_ref[...])], axis=1)


def _a_map(t):
    return (0, jnp.where(t == _NB, 2 * _NB, t))


def _b_map(t):
    return (0, jnp.where(t == _NB, 0, t + _NB))


_tc_transpose = pl.pallas_call(
    _transpose_block,
    grid=(_NB + 1,),
    in_specs=[
        pl.BlockSpec((EMBEDDING_DIM, EMBEDDING_DIM), lambda t: (0, 0)),
        pl.BlockSpec((EMBEDDING_DIM, _TW), _a_map),
        pl.BlockSpec((EMBEDDING_DIM, _TW), _b_map),
    ],
    out_specs=pl.BlockSpec((_TW, 2 * EMBEDDING_DIM), lambda t: (t, 0)),
    out_shape=jax.ShapeDtypeStruct((_PK_ROWS, 2 * EMBEDDING_DIM), jnp.float32),
)


def _make_gather():
    mesh = plsc.VectorSubcoreMesh(core_axis_name="c", subcore_axis_name="s")

    @functools.partial(
        pl.kernel,
        mesh=mesh,
        out_type=jax.ShapeDtypeStruct((BATCH, EMBEDDING_DIM), jnp.float32),
        scratch_types=[
            pltpu.VMEM((_B_PER_W,), jnp.int32),
            pltpu.SMEM((_B_PER_W,), jnp.int32),
            pltpu.VMEM((2, _C_ROWS, 2 * EMBEDDING_DIM), jnp.float32),
            pltpu.VMEM((_C_ROWS, EMBEDDING_DIM), jnp.float32),
            [pltpu.SemaphoreType.DMA] * 2,
        ],
    )
    def gather_kernel(table_hbm, idx_hbm, out_hbm, idx_v, q_s, rbuf, obuf, sems):
        wid = lax.axis_index("s") * _NC + lax.axis_index("c")
        base = wid * _B_PER_W
        pltpu.sync_copy(idx_hbm.at[pl.ds(base, _B_PER_W)], idx_v)

        def issue_chunk(j):
            p = j % 2
            for h in range(_C_ROWS // _L):
                v = idx_v[pl.ds(j * _C_ROWS + h * _L, _L)]
                for l in range(_L):
                    x = v[l]
                    in_b = jnp.logical_and(x >= _H, x < 2 * _H)
                    row = jnp.where(x >= _H, x - _H, x)
                    q_s[j * _C_ROWS + h * _L + l] = in_b.astype(jnp.int32)
                    pltpu.async_copy(
                        table_hbm.at[row], rbuf.at[p, h * _L + l], sems[p]
                    )

        def drain_select_writeback(j):
            p = j % 2
            dst = out_hbm.at[pl.ds(base + j * _C_ROWS, _C_ROWS)]
            pltpu.make_async_copy(
                table_hbm.at[pl.ds(0, _C_ROWS)], rbuf.at[p], sems[p]
            ).wait()

            def row_body(i, carry, j=j, p=p):
                q = q_s[j * _C_ROWS + i]

                @pl.when(q == 0)
                def _():
                    for k in range(EMBEDDING_DIM // _L):
                        obuf[i, pl.ds(k * _L, _L)] = rbuf[p, i, pl.ds(k * _L, _L)]

                @pl.when(q == 1)
                def _():
                    for k in range(EMBEDDING_DIM // _L):
                        obuf[i, pl.ds(k * _L, _L)] = rbuf[
                            p, i, pl.ds(EMBEDDING_DIM + k * _L, _L)
                        ]

                return carry

            lax.fori_loop(0, _C_ROWS, row_body, None)
            pltpu.sync_copy(obuf, dst)

        issue_chunk(0)
        for j in range(1, _NCHUNK):
            issue_chunk(j)
            drain_select_writeback(j - 1)
        drain_select_writeback(_NCHUNK - 1)

    return gather_kernel


_gather = _make_gather()


def kernel(state_id, state_embedding):
    eye = jnp.eye(EMBEDDING_DIM, dtype=jnp.bfloat16)
    t_t = state_embedding.T
    table_pk = _tc_transpose(eye, t_t, t_t)
    return _gather(table_pk, state_id.astype(jnp.int32))
